# Initial kernel scaffold; baseline (speedup 1.0000x reference)
import functools

import jax
import jax.numpy as jnp
from jax import lax
from jax.experimental import pallas as pl
from jax.experimental.pallas import tpu as pltpu
from jax.experimental.pallas import tpu_sc as plsc

B, S, V, D, C = 4096, 200, 100000, 100, 90
NC, NS = 2, 16
NW = NC * NS
BPW = B // NW
HALF = S // 2
DPAD = 112
NCH = DPAD // 16


def _pool_body(ids_hbm, table_hbm, out_hbm, idx_v, g_v, out_v, gsem):
    w = lax.axis_index("s") * NC + lax.axis_index("c")
    pltpu.sync_copy(ids_hbm.at[pl.ds(w * 2 * BPW, 2 * BPW)], idx_v)
    lane = lax.broadcasted_iota(jnp.int32, (16,), 0)
    tail_col = jnp.minimum(96 + lane, D - 1)
    tail_mask = lane < (D - 96)
    zero = jnp.zeros((16,), jnp.float32)

    def row_step(b, _):
        cp0 = pltpu.async_copy(
            table_hbm.at[idx_v.at[2 * b]], g_v.at[pl.ds(0, HALF)], gsem)
        cp1 = pltpu.async_copy(
            table_hbm.at[idx_v.at[2 * b + 1]], g_v.at[pl.ds(HALF, HALF)], gsem)
        cp0.wait()
        cp1.wait()

        def tok_step(r, acc):
            new = [acc[j] + g_v[r, pl.ds(j * 16, 16)] for j in range(6)]
            x = plsc.load_gather(
                g_v, [jnp.full((16,), r, jnp.int32), tail_col])
            new.append(acc[6] + jnp.where(tail_mask, x, 0.0))
            return tuple(new)

        acc = lax.fori_loop(0, S, tok_step, (zero,) * NCH)
        for j in range(NCH):
            out_v[b, pl.ds(j * 16, 16)] = acc[j]
        return 0

    lax.fori_loop(0, BPW, row_step, 0)
    pltpu.sync_copy(out_v, out_hbm.at[pl.ds(w * BPW, BPW)])


_pool = functools.partial(
    pl.kernel,
    out_type=jax.ShapeDtypeStruct((B, DPAD), jnp.float32),
    mesh=plsc.VectorSubcoreMesh(core_axis_name="c", subcore_axis_name="s"),
    scratch_types=[
        pltpu.VMEM((2 * BPW, HALF), jnp.int32),
        pltpu.VMEM((S, D), jnp.float32),
        pltpu.VMEM((BPW, DPAD), jnp.float32),
        pltpu.SemaphoreType.DMA,
    ],
)(_pool_body)


def _mm_body(p_ref, w_ref, b_ref, o_ref):
    o_ref[...] = (
        jnp.dot(p_ref[...], w_ref[...], preferred_element_type=jnp.float32)
        * (1.0 / S) + b_ref[...])


def kernel(input_ids, embedding, fc_w, fc_b):
    ids2 = input_ids.reshape(2 * B, HALF)
    sums = _pool(ids2, embedding)
    fc_w_pad = jnp.concatenate(
        [fc_w, jnp.zeros((DPAD - D, C), jnp.float32)], axis=0)
    gb = 512
    return pl.pallas_call(
        _mm_body,
        grid=(B // gb,),
        in_specs=[
            pl.BlockSpec((gb, DPAD), lambda i: (i, 0)),
            pl.BlockSpec((DPAD, C), lambda i: (0, 0)),
            pl.BlockSpec((1, C), lambda i: (0, 0)),
        ],
        out_specs=pl.BlockSpec((gb, C), lambda i: (i, 0)),
        out_shape=jax.ShapeDtypeStruct((B, C), jnp.float32),
    )(sums, fc_w_pad, fc_b.reshape(1, C))


# speedup vs baseline: 5.2529x; 5.2529x over previous
import functools

import jax
import jax.numpy as jnp
from jax import lax
from jax.experimental import pallas as pl
from jax.experimental.pallas import tpu as pltpu
from jax.experimental.pallas import tpu_sc as plsc

B, S, V, D, C = 4096, 200, 100000, 100, 90
NC, NS = 2, 16
NW = NC * NS
BPW = B // NW
HALF = S // 2
DPAD = 128
NCH = DPAD // 16


def _pool_body(ids_hbm, table_hbm, out_hbm, idx_v, g_v, out_v, gsem):
    w = lax.axis_index("s") * NC + lax.axis_index("c")
    pltpu.sync_copy(ids_hbm.at[pl.ds(w * 2 * BPW, 2 * BPW)], idx_v)
    zero = jnp.zeros((16,), jnp.float32)

    def row_step(b, _):
        cp0 = pltpu.async_copy(
            table_hbm.at[idx_v.at[2 * b]], g_v.at[pl.ds(0, HALF)], gsem)
        cp1 = pltpu.async_copy(
            table_hbm.at[idx_v.at[2 * b + 1]], g_v.at[pl.ds(HALF, HALF)], gsem)
        cp0.wait()
        cp1.wait()

        def tok_step(r, acc):
            return tuple(
                acc[j] + g_v[r, pl.ds(j * 16, 16)] for j in range(NCH))

        acc = lax.fori_loop(0, S, tok_step, (zero,) * NCH)
        for j in range(NCH):
            out_v[b, pl.ds(j * 16, 16)] = acc[j]
        return 0

    lax.fori_loop(0, BPW, row_step, 0)
    pltpu.sync_copy(out_v, out_hbm.at[pl.ds(w * BPW, BPW)])


_pool = functools.partial(
    pl.kernel,
    out_type=jax.ShapeDtypeStruct((B, DPAD), jnp.float32),
    mesh=plsc.VectorSubcoreMesh(core_axis_name="c", subcore_axis_name="s"),
    scratch_types=[
        pltpu.VMEM((2 * BPW, HALF), jnp.int32),
        pltpu.VMEM((S, DPAD), jnp.float32),
        pltpu.VMEM((BPW, DPAD), jnp.float32),
        pltpu.SemaphoreType.DMA,
    ],
)(_pool_body)


def _mm_body(p_ref, w_ref, b_ref, o_ref):
    o_ref[...] = (
        jnp.dot(p_ref[...], w_ref[...], preferred_element_type=jnp.float32)
        * (1.0 / S) + b_ref[...])


def kernel(input_ids, embedding, fc_w, fc_b):
    ids2 = input_ids.reshape(2 * B, HALF)
    table_pad = jnp.pad(embedding, ((0, 0), (0, DPAD - D)))
    sums = _pool(ids2, table_pad)
    fc_w_pad = jnp.pad(fc_w, ((0, DPAD - D), (0, 0)))
    gb = 512
    return pl.pallas_call(
        _mm_body,
        grid=(B // gb,),
        in_specs=[
            pl.BlockSpec((gb, DPAD), lambda i: (i, 0)),
            pl.BlockSpec((DPAD, C), lambda i: (0, 0)),
            pl.BlockSpec((1, C), lambda i: (0, 0)),
        ],
        out_specs=pl.BlockSpec((gb, C), lambda i: (i, 0)),
        out_shape=jax.ShapeDtypeStruct((B, C), jnp.float32),
    )(sums, fc_w_pad, fc_b.reshape(1, C))


# speedup vs baseline: 7.3092x; 1.3915x over previous
import functools

import jax
import jax.numpy as jnp
from jax import lax
from jax.experimental import pallas as pl
from jax.experimental.pallas import tpu as pltpu
from jax.experimental.pallas import tpu_sc as plsc

B, S, V, D, C = 4096, 200, 100000, 100, 90
NC, NS = 2, 16
NW = NC * NS
BPW = B // NW
HALF = S // 2
DPAD = 128
NCH = DPAD // 16


def _pool_body(ids_hbm, table_hbm, out_hbm, idx_v, g_v, out_v, sem0, sem1):
    w = lax.axis_index("s") * NC + lax.axis_index("c")
    pltpu.sync_copy(ids_hbm.at[pl.ds(w * 2 * BPW, 2 * BPW)], idx_v)
    zero = jnp.zeros((16,), jnp.float32)
    sems = (sem0, sem1)

    def issue(b, buf, sem):
        pltpu.async_copy(
            table_hbm.at[idx_v.at[2 * b]], g_v.at[buf, pl.ds(0, HALF)], sem)
        pltpu.async_copy(
            table_hbm.at[idx_v.at[2 * b + 1]],
            g_v.at[buf, pl.ds(HALF, HALF)], sem)

    def drain(b, buf, sem):
        pltpu.make_async_copy(
            table_hbm.at[idx_v.at[2 * b]],
            g_v.at[buf, pl.ds(0, HALF)], sem).wait()
        pltpu.make_async_copy(
            table_hbm.at[idx_v.at[2 * b + 1]],
            g_v.at[buf, pl.ds(HALF, HALF)], sem).wait()

    def fold(b, buf):
        def tok_step(r, acc):
            return tuple(
                acc[j] + g_v[buf, r, pl.ds(j * 16, 16)] for j in range(NCH))

        acc = lax.fori_loop(0, S, tok_step, (zero,) * NCH)
        for j in range(NCH):
            out_v[b, pl.ds(j * 16, 16)] = acc[j]

    issue(0, 0, sems[0])

    def pair_step(bp, _):
        b0 = 2 * bp
        b1 = b0 + 1
        issue(b1, 1, sems[1])
        drain(b0, 0, sems[0])
        fold(b0, 0)

        @pl.when(bp < BPW // 2 - 1)
        def _():
            issue(b0 + 2, 0, sems[0])

        drain(b1, 1, sems[1])
        fold(b1, 1)
        return 0

    lax.fori_loop(0, BPW // 2, pair_step, 0)
    pltpu.sync_copy(out_v, out_hbm.at[pl.ds(w * BPW, BPW)])


_pool = functools.partial(
    pl.kernel,
    out_type=jax.ShapeDtypeStruct((B, DPAD), jnp.float32),
    mesh=plsc.VectorSubcoreMesh(core_axis_name="c", subcore_axis_name="s"),
    scratch_types=[
        pltpu.VMEM((2 * BPW, HALF), jnp.int32),
        pltpu.VMEM((2, S, DPAD), jnp.float32),
        pltpu.VMEM((BPW, DPAD), jnp.float32),
        pltpu.SemaphoreType.DMA,
        pltpu.SemaphoreType.DMA,
    ],
)(_pool_body)


def _mm_body(p_ref, w_ref, b_ref, o_ref):
    o_ref[...] = (
        jnp.dot(p_ref[...], w_ref[...], preferred_element_type=jnp.float32)
        * (1.0 / S) + b_ref[...])


def kernel(input_ids, embedding, fc_w, fc_b):
    ids2 = input_ids.reshape(2 * B, HALF)
    table_pad = jnp.pad(embedding, ((0, 0), (0, DPAD - D)))
    sums = _pool(ids2, table_pad)
    fc_w_pad = jnp.pad(fc_w, ((0, DPAD - D), (0, 0)))
    gb = 512
    return pl.pallas_call(
        _mm_body,
        grid=(B // gb,),
        in_specs=[
            pl.BlockSpec((gb, DPAD), lambda i: (i, 0)),
            pl.BlockSpec((DPAD, C), lambda i: (0, 0)),
            pl.BlockSpec((1, C), lambda i: (0, 0)),
        ],
        out_specs=pl.BlockSpec((gb, C), lambda i: (i, 0)),
        out_shape=jax.ShapeDtypeStruct((B, C), jnp.float32),
    )(sums, fc_w_pad, fc_b.reshape(1, C))


# speedup vs baseline: 10.3747x; 1.4194x over previous
import functools

import jax
import jax.numpy as jnp
from jax import lax
from jax.experimental import pallas as pl
from jax.experimental.pallas import tpu as pltpu
from jax.experimental.pallas import tpu_sc as plsc

B, S, V, D, C = 4096, 200, 100000, 100, 90
NC, NS = 2, 16
NW = NC * NS
BPW = B // NW
HALF = S // 2
DPAD = 128
NCH = DPAD // 16


def _pool_body(ids_hbm, table_hbm, out_hbm, idx_v, g_v, out_v, sem0, sem1):
    w = lax.axis_index("s") * NC + lax.axis_index("c")
    pltpu.sync_copy(ids_hbm.at[pl.ds(w * 2 * BPW, 2 * BPW)], idx_v)
    zero = jnp.zeros((16,), jnp.float32)
    sems = (sem0, sem1)

    def issue(b, buf, sem):
        pltpu.async_copy(
            table_hbm.at[idx_v.at[2 * b]], g_v.at[buf, pl.ds(0, HALF)], sem)
        pltpu.async_copy(
            table_hbm.at[idx_v.at[2 * b + 1]],
            g_v.at[buf, pl.ds(HALF, HALF)], sem)

    def drain(b, buf, sem):
        pltpu.make_async_copy(
            table_hbm.at[idx_v.at[2 * b]],
            g_v.at[buf, pl.ds(0, HALF)], sem).wait()
        pltpu.make_async_copy(
            table_hbm.at[idx_v.at[2 * b + 1]],
            g_v.at[buf, pl.ds(HALF, HALF)], sem).wait()

    def fold(b, buf):
        def tok_step(r, acc):
            return tuple(
                acc[j] + g_v[buf, r, pl.ds(j * 16, 16)] for j in range(NCH))

        acc = lax.fori_loop(0, S, tok_step, (zero,) * NCH)
        for j in range(NCH):
            out_v[b, pl.ds(j * 16, 16)] = acc[j]

    issue(0, 0, sems[0])

    def pair_step(bp, _):
        b0 = 2 * bp
        b1 = b0 + 1
        issue(b1, 1, sems[1])
        drain(b0, 0, sems[0])
        fold(b0, 0)

        @pl.when(bp < BPW // 2 - 1)
        def _():
            issue(b0 + 2, 0, sems[0])

        drain(b1, 1, sems[1])
        fold(b1, 1)
        return 0

    lax.fori_loop(0, BPW // 2, pair_step, 0)
    pltpu.sync_copy(out_v, out_hbm.at[pl.ds(w * BPW, BPW)])


_pool = functools.partial(
    pl.kernel,
    out_type=jax.ShapeDtypeStruct((B, DPAD), jnp.float32),
    mesh=plsc.VectorSubcoreMesh(core_axis_name="c", subcore_axis_name="s"),
    scratch_types=[
        pltpu.VMEM((2 * BPW, HALF), jnp.int32),
        pltpu.VMEM((2, S, DPAD), jnp.float32),
        pltpu.VMEM((BPW, DPAD), jnp.float32),
        pltpu.SemaphoreType.DMA,
        pltpu.SemaphoreType.DMA,
    ],
)(_pool_body)


def _pad_body(x_ref, o_ref):
    x = x_ref[...]
    o_ref[...] = jnp.concatenate(
        [x, jnp.zeros((x.shape[0], DPAD - D), jnp.float32)], axis=-1)


def _pad_table(embedding):
    rblk = 20000
    return pl.pallas_call(
        _pad_body,
        grid=(V // rblk,),
        in_specs=[pl.BlockSpec((rblk, D), lambda i: (i, 0))],
        out_specs=pl.BlockSpec((rblk, DPAD), lambda i: (i, 0)),
        out_shape=jax.ShapeDtypeStruct((V, DPAD), jnp.float32),
    )(embedding)


def _mm_body(p_ref, w_ref, b_ref, o_ref):
    o_ref[...] = (
        jnp.dot(p_ref[...], w_ref[...], preferred_element_type=jnp.float32)
        * (1.0 / S) + b_ref[...])


def kernel(input_ids, embedding, fc_w, fc_b):
    ids2 = input_ids.reshape(2 * B, HALF)
    table_pad = _pad_table(embedding)
    sums = _pool(ids2, table_pad)
    fc_w_pad = jnp.pad(fc_w, ((0, DPAD - D), (0, 0)))
    gb = 512
    return pl.pallas_call(
        _mm_body,
        grid=(B // gb,),
        in_specs=[
            pl.BlockSpec((gb, DPAD), lambda i: (i, 0)),
            pl.BlockSpec((DPAD, C), lambda i: (0, 0)),
            pl.BlockSpec((1, C), lambda i: (0, 0)),
        ],
        out_specs=pl.BlockSpec((gb, C), lambda i: (i, 0)),
        out_shape=jax.ShapeDtypeStruct((B, C), jnp.float32),
    )(sums, fc_w_pad, fc_b.reshape(1, C))
